# baseline (speedup 1.0000x reference)
import functools

import jax
import jax.numpy as jnp
from jax import lax
from jax.experimental import pallas as pl
from jax.experimental.pallas import tpu as pltpu
from jax.experimental.pallas import tpu_sc as plsc

VOCAB_ROWS = 1000000
EMBED_DIM = 64
BATCH = 16384
N_FIELDS = 26
FPAD = 32
OUT_ROWS = BATCH * FPAD

NUM_CORES = 2
NUM_SUBCORES = 16
NW = NUM_CORES * NUM_SUBCORES
B_PER_W = BATCH // NW
PER_W = B_PER_W * N_FIELDS
IDX_ROWS = PER_W // 128
G = 16
N_CHUNKS = B_PER_W // G
CROWS = G * N_FIELDS

_mesh = plsc.VectorSubcoreMesh(core_axis_name="c", subcore_axis_name="s")


@functools.partial(
    pl.kernel,
    mesh=_mesh,
    out_type=jax.ShapeDtypeStruct((OUT_ROWS, 128), jnp.float32),
    scratch_types=[
        pltpu.VMEM((B_PER_W, FPAD), jnp.int32),
        pltpu.VMEM((G * FPAD, EMBED_DIM), jnp.float32),
        pltpu.VMEM((G * FPAD, EMBED_DIM), jnp.float32),
        pltpu.SemaphoreType.DMA,
        pltpu.SemaphoreType.DMA,
        pltpu.SemaphoreType.DMA,
        pltpu.SemaphoreType.DMA,
    ],
    compiler_params=pltpu.CompilerParams(use_tc_tiling_on_sc=False),
)
def _gather_k(idx_hbm, table_hbm, out_hbm, idx_v, buf0, buf1, g0, g1, s0, s1):
    wid = lax.axis_index("s") * NUM_CORES + lax.axis_index("c")
    base = wid * B_PER_W
    pltpu.sync_copy(idx_hbm.at[pl.ds(base, B_PER_W), pl.ds(0, FPAD)], idx_v)

    bufs = (buf0, buf1)
    gsems = (g0, g1)
    ssems = (s0, s1)

    def start_gather(c, buf, gsem):
        def row(r, carry):
            pltpu.async_copy(
                table_hbm.at[idx_v.at[c * G + r]],
                buf.at[pl.ds(r * FPAD, FPAD)],
                gsem,
            )
            return carry

        lax.fori_loop(0, G, row, 0)

    def drain(buf, sem):
        pltpu.make_async_copy(table_hbm.at[pl.ds(0, G * FPAD)], buf, sem).wait()

    def start_store(c, buf, ssem):
        def row(r, carry):
            b = base + c * G + r
            pltpu.async_copy(
                buf.at[pl.ds(r * FPAD, N_FIELDS)],
                out_hbm.at[pl.ds(b * FPAD, N_FIELDS), pl.ds(0, EMBED_DIM)],
                ssem,
            )
            return carry

        lax.fori_loop(0, G, row, 0)

    def drain_store(c, buf, ssem):
        def row(r, carry):
            b = base + c * G + r
            pltpu.make_async_copy(
                buf.at[pl.ds(r * FPAD, N_FIELDS)],
                out_hbm.at[pl.ds(b * FPAD, N_FIELDS), pl.ds(0, EMBED_DIM)],
                ssem,
            ).wait()
            return carry

        lax.fori_loop(0, G, row, 0)

    start_gather(0, buf0, g0)
    start_gather(1, buf1, g1)

    def body(i, carry):
        for p in range(2):
            c = i * 2 + p
            buf, gsem, ssem = bufs[p], gsems[p], ssems[p]
            drain(buf, gsem)
            start_store(c, buf, ssem)

            @pl.when(c + 2 < N_CHUNKS)
            def _():
                drain_store(c, buf, ssem)
                start_gather(c + 2, buf, gsem)

        return carry

    lax.fori_loop(0, N_CHUNKS // 2, body, 0)
    drain_store(N_CHUNKS - 2, buf0, s0)
    drain_store(N_CHUNKS - 1, buf1, s1)


def kernel(index, cluster_index):
    idxp = jnp.pad(index, ((0, 0), (0, 128 - N_FIELDS)))
    out = _gather_k(idxp, cluster_index)
    return out.reshape(BATCH, FPAD, 128)[:, :N_FIELDS, :EMBED_DIM]


# speedup vs baseline: 3.5930x; 3.5930x over previous
import functools

import jax
import jax.numpy as jnp
from jax import lax
from jax.experimental import pallas as pl
from jax.experimental.pallas import tpu as pltpu
from jax.experimental.pallas import tpu_sc as plsc

VOCAB_ROWS = 1000000
EMBED_DIM = 64
BATCH = 16384
N_FIELDS = 26
FPAD = 32
OUT_ROWS = BATCH * FPAD

NUM_CORES = 2
NUM_SUBCORES = 16
NW = NUM_CORES * NUM_SUBCORES
B_PER_W = BATCH // NW
PER_W = B_PER_W * N_FIELDS
IDX_ROWS = PER_W // 128
G = 16
N_CHUNKS = B_PER_W // G
CROWS = G * N_FIELDS

_mesh = plsc.VectorSubcoreMesh(core_axis_name="c", subcore_axis_name="s")


@functools.partial(
    pl.kernel,
    mesh=_mesh,
    out_type=jax.ShapeDtypeStruct((OUT_ROWS, 128), jnp.float32),
    scratch_types=[
        pltpu.VMEM((PER_W,), jnp.int32),
        pltpu.VMEM((CROWS, EMBED_DIM), jnp.float32),
        pltpu.VMEM((CROWS, EMBED_DIM), jnp.float32),
        pltpu.SemaphoreType.DMA,
        pltpu.SemaphoreType.DMA,
        pltpu.SemaphoreType.DMA,
        pltpu.SemaphoreType.DMA,
    ],
    compiler_params=pltpu.CompilerParams(use_tc_tiling_on_sc=False),
)
def _gather_k(table_hbm, idx_hbm, out_hbm, idx_v, buf0, buf1, g0, g1, s0, s1):
    wid = lax.axis_index("s") * NUM_CORES + lax.axis_index("c")
    base = wid * B_PER_W

    def flat_row(k, carry):
        pltpu.async_copy(idx_hbm.at[wid, k], idx_v.at[pl.ds(k * 128, 128)], g0)
        return carry

    lax.fori_loop(0, IDX_ROWS, flat_row, 0)

    def flat_drain(k, carry):
        pltpu.make_async_copy(
            idx_hbm.at[wid, k], idx_v.at[pl.ds(k * 128, 128)], g0
        ).wait()
        return carry

    lax.fori_loop(0, IDX_ROWS, flat_drain, 0)

    bufs = (buf0, buf1)
    gsems = (g0, g1)
    ssems = (s0, s1)

    def start_gather(c, buf, gsem):
        pltpu.async_copy(
            table_hbm.at[idx_v.at[pl.ds(c * CROWS, CROWS)]], buf, gsem
        )

    def drain(buf, sem):
        pltpu.make_async_copy(table_hbm.at[pl.ds(0, CROWS)], buf, sem).wait()

    def start_store(c, buf, ssem):
        def row(r, carry):
            b = base + c * G + r
            pltpu.async_copy(
                buf.at[pl.ds(r * N_FIELDS, N_FIELDS)],
                out_hbm.at[pl.ds(b * FPAD, N_FIELDS), pl.ds(0, EMBED_DIM)],
                ssem,
            )
            return carry

        lax.fori_loop(0, G, row, 0)

    start_gather(0, buf0, g0)
    start_gather(1, buf1, g1)

    def body(i, carry):
        for p in range(2):
            c = i * 2 + p
            buf, gsem, ssem = bufs[p], gsems[p], ssems[p]
            drain(buf, gsem)
            start_store(c, buf, ssem)

            @pl.when(c + 2 < N_CHUNKS)
            def _():
                drain(buf, ssem)
                start_gather(c + 2, buf, gsem)

        return carry

    lax.fori_loop(0, N_CHUNKS // 2, body, 0)
    drain(buf0, s0)
    drain(buf1, s1)


def kernel(index, cluster_index):
    idx3 = index.reshape(NW, IDX_ROWS, 128)
    out = _gather_k(cluster_index, idx3)
    return out.reshape(BATCH, FPAD, 128)[:, :N_FIELDS, :EMBED_DIM]
